# Initial kernel scaffold; baseline (speedup 1.0000x reference)
import jax
import jax.numpy as jnp
from jax.experimental import pallas as pl


def kernel(train_mat):
    raise NotImplementedError("write your pallas kernel here")


# speedup vs baseline: 1.2308x; 1.2308x over previous
import functools

import jax
import jax.numpy as jnp
from jax.experimental import pallas as pl
from jax.experimental.pallas import tpu as pltpu

_U = 2048
_I = 16384
_K = 10
_THR = 0.2
_BU = 256
_NB = _U // _BU
_IC = _I // 8


def _rowsum_kernel(tb_ref, out_ref):
    out_ref[...] = jnp.sum(tb_ref[...].astype(jnp.float32), axis=1,
                           keepdims=True)


def _jac_kernel(a_ref, bt_ref, rcol_ref, rrow_ref, out_ref):
    i = pl.program_id(0)
    j = pl.program_id(1)
    num = jnp.dot(a_ref[...], bt_ref[...],
                  preferred_element_type=jnp.float32)
    den = rcol_ref[...] + rrow_ref[...] - num
    den = jnp.where(den == 0.0, 1.0, den)
    jac = num / den
    rows = i * _BU + jax.lax.broadcasted_iota(jnp.int32, (_BU, _BU), 0)
    cols = j * _BU + jax.lax.broadcasted_iota(jnp.int32, (_BU, _BU), 1)
    out_ref[...] = jnp.where(rows == cols, 0.0, jac)


def _topkw_kernel(jac_ref, w_ref):
    x = jac_ref[...]
    mask = (x > _THR).astype(jnp.float32)
    counts = jnp.sum(mask, axis=1, keepdims=True)
    iota = jax.lax.broadcasted_iota(jnp.int32, x.shape, 1)
    acc = jnp.zeros_like(x)
    for _ in range(_K):
        m = jnp.max(x, axis=1, keepdims=True)
        first = jnp.min(jnp.where(x == m, iota, _U), axis=1, keepdims=True)
        onehot = iota == first
        acc = jnp.where(onehot, 1.0, acc)
        x = jnp.where(onehot, -1.0, x)
    w_ref[...] = jnp.where(counts >= float(_K), mask, acc).astype(jnp.bfloat16)


def _d_kernel(w_ref, tb_ref, tf_ref, d_ref, db_ref):
    w = w_ref[...]
    wsum = jnp.sum(w.astype(jnp.float32), axis=1, keepdims=True)
    nm = jnp.dot(w, tb_ref[...], preferred_element_type=jnp.float32)
    nm = nm / jnp.maximum(wsum, 1.0)
    d = 0.5 * tf_ref[...] + 0.5 * nm
    d_ref[...] = d
    db_ref[...] = d.astype(jnp.bfloat16)


def _norm_kernel(d_ref, n_ref):
    s = jnp.sum(d_ref[...] ** 2, axis=1, keepdims=True)
    n_ref[...] = jnp.maximum(jnp.sqrt(s), 1e-12)


def _cos_kernel(a_ref, bt_ref, ncol_ref, nrow_ref, out_ref, acc_ref):
    k = pl.program_id(2)

    @pl.when(k == 0)
    def _init():
        acc_ref[...] = jnp.zeros_like(acc_ref)

    acc_ref[...] += jnp.dot(a_ref[...], bt_ref[...],
                            preferred_element_type=jnp.float32)

    @pl.when(k == pl.num_programs(2) - 1)
    def _fin():
        out_ref[...] = acc_ref[...] / ncol_ref[...] / nrow_ref[...]


def _topk2_kernel(c_ref, vals_ref, idx_ref):
    x = c_ref[...]
    iota = jax.lax.broadcasted_iota(jnp.int32, x.shape, 1)
    for t in range(_K):
        m = jnp.max(x, axis=1, keepdims=True)
        first = jnp.min(jnp.where(x == m, iota, _U), axis=1, keepdims=True)
        onehot = iota == first
        vals_ref[:, t:t + 1] = m
        idx_ref[:, t:t + 1] = first
        x = jnp.where(onehot, -1.0, x)


def kernel(train_mat):
    f32 = jnp.float32
    tb = train_mat.astype(jnp.bfloat16)
    tbt = tb.T

    r = pl.pallas_call(
        _rowsum_kernel,
        grid=(_NB,),
        in_specs=[pl.BlockSpec((_BU, _I), lambda i: (i, 0))],
        out_specs=pl.BlockSpec((_BU, 1), lambda i: (i, 0)),
        out_shape=jax.ShapeDtypeStruct((_U, 1), f32),
    )(tb)
    rrow = r.reshape(1, _U)

    jac = pl.pallas_call(
        _jac_kernel,
        grid=(_NB, _NB),
        in_specs=[
            pl.BlockSpec((_BU, _I), lambda i, j: (i, 0)),
            pl.BlockSpec((_I, _BU), lambda i, j: (0, j)),
            pl.BlockSpec((_BU, 1), lambda i, j: (i, 0)),
            pl.BlockSpec((1, _BU), lambda i, j: (0, j)),
        ],
        out_specs=pl.BlockSpec((_BU, _BU), lambda i, j: (i, j)),
        out_shape=jax.ShapeDtypeStruct((_U, _U), f32),
    )(tb, tbt, r, rrow)

    w = pl.pallas_call(
        _topkw_kernel,
        grid=(_NB,),
        in_specs=[pl.BlockSpec((_BU, _U), lambda i: (i, 0))],
        out_specs=pl.BlockSpec((_BU, _U), lambda i: (i, 0)),
        out_shape=jax.ShapeDtypeStruct((_U, _U), jnp.bfloat16),
    )(jac)

    d, db = pl.pallas_call(
        _d_kernel,
        grid=(8, _NB),
        in_specs=[
            pl.BlockSpec((_BU, _U), lambda j, i: (i, 0)),
            pl.BlockSpec((_U, _IC), lambda j, i: (0, j)),
            pl.BlockSpec((_BU, _IC), lambda j, i: (i, j)),
        ],
        out_specs=[
            pl.BlockSpec((_BU, _IC), lambda j, i: (i, j)),
            pl.BlockSpec((_BU, _IC), lambda j, i: (i, j)),
        ],
        out_shape=[
            jax.ShapeDtypeStruct((_U, _I), f32),
            jax.ShapeDtypeStruct((_U, _I), jnp.bfloat16),
        ],
    )(w, tb, train_mat)

    n = pl.pallas_call(
        _norm_kernel,
        grid=(_NB,),
        in_specs=[pl.BlockSpec((_BU, _I), lambda i: (i, 0))],
        out_specs=pl.BlockSpec((_BU, 1), lambda i: (i, 0)),
        out_shape=jax.ShapeDtypeStruct((_U, 1), f32),
    )(d)
    nrow = n.reshape(1, _U)

    dbt = d.T
    kc = _I // 4
    cos = pl.pallas_call(
        _cos_kernel,
        grid=(_NB, _NB, 4),
        in_specs=[
            pl.BlockSpec((_BU, kc), lambda i, j, k: (i, k)),
            pl.BlockSpec((kc, _BU), lambda i, j, k: (k, j)),
            pl.BlockSpec((_BU, 1), lambda i, j, k: (i, 0)),
            pl.BlockSpec((1, _BU), lambda i, j, k: (0, j)),
        ],
        out_specs=pl.BlockSpec((_BU, _BU), lambda i, j, k: (i, j)),
        out_shape=jax.ShapeDtypeStruct((_U, _U), f32),
        scratch_shapes=[pltpu.VMEM((_BU, _BU), f32)],
    )(d, dbt, n, nrow)

    vals, idx = pl.pallas_call(
        _topk2_kernel,
        grid=(_NB,),
        in_specs=[pl.BlockSpec((_BU, _U), lambda i: (i, 0))],
        out_specs=[
            pl.BlockSpec((_BU, _K), lambda i: (i, 0)),
            pl.BlockSpec((_BU, _K), lambda i: (i, 0)),
        ],
        out_shape=[
            jax.ShapeDtypeStruct((_U, _K), f32),
            jax.ShapeDtypeStruct((_U, _K), jnp.int32),
        ],
    )(cos)
    return vals, idx


# speedup vs baseline: 1.9081x; 1.5503x over previous
import functools

import jax
import jax.numpy as jnp
from jax.experimental import pallas as pl
from jax.experimental.pallas import tpu as pltpu

_U = 2048
_I = 16384
_K = 10
_THR = 0.2
_BU = 256
_NB = _U // _BU
_BC = 512
_NC = _U // _BC
_KC = _I // 4
_IC = _I // 8


def _tri_ij(t, n):
    i = jnp.zeros((), jnp.int32)
    off = 0
    for m in range(1, n):
        off += n - (m - 1)
        i = i + (t >= off).astype(jnp.int32)
    offs_i = i * n - (i * (i - 1)) // 2
    j = t - offs_i + i
    return i, j


def _rowsum_kernel(tb_ref, out_ref):
    out_ref[...] = jnp.sum(tb_ref[...].astype(jnp.float32), axis=1,
                           keepdims=True)


def _jacnum_kernel(a_ref, bt_ref, up_ref, lo_ref):
    num = jnp.dot(a_ref[...], bt_ref[...],
                  preferred_element_type=jnp.float32)
    up_ref[...] = num
    lo_ref[...] = num.T


def _topk_sweep(x, iota):
    vals, idxs = [], []
    for _ in range(_K):
        m = jnp.max(x, axis=1, keepdims=True)
        first = jnp.min(jnp.where(x == m, iota, _U), axis=1, keepdims=True)
        onehot = iota == first
        vals.append(m)
        idxs.append(first)
        x = jnp.where(onehot, -1.0, x)
    return vals, idxs


def _topkw_kernel(up_ref, lo_ref, rcol_ref, rrow_ref, w_ref):
    i = pl.program_id(0)
    cols = jax.lax.broadcasted_iota(jnp.int32, (_BU, _U), 1)
    num = jnp.where(cols >= i * _BU, up_ref[...], lo_ref[...])
    den = rcol_ref[...] + rrow_ref[...] - num
    den = jnp.where(den == 0.0, 1.0, den)
    jacv = num / den
    rows = i * _BU + jax.lax.broadcasted_iota(jnp.int32, (_BU, _U), 0)
    x = jnp.where(rows == cols, 0.0, jacv)
    mask = (x > _THR).astype(jnp.float32)
    counts = jnp.sum(mask, axis=1, keepdims=True)
    acc = jnp.zeros_like(x)
    y = x
    for _ in range(_K):
        m = jnp.max(y, axis=1, keepdims=True)
        first = jnp.min(jnp.where(y == m, cols, _U), axis=1, keepdims=True)
        onehot = cols == first
        acc = jnp.where(onehot, 1.0, acc)
        y = jnp.where(onehot, -1.0, y)
    w_ref[...] = jnp.where(counts >= float(_K), mask, acc).astype(jnp.bfloat16)


def _d_kernel(w_ref, tb_ref, trow_ref, d_ref, p_ref):
    w = w_ref[...]
    wsum = jnp.sum(w.astype(jnp.float32), axis=1, keepdims=True)
    nm = jnp.dot(w, tb_ref[...], preferred_element_type=jnp.float32)
    nm = nm / jnp.maximum(wsum, 1.0)
    d = 0.5 * trow_ref[...].astype(jnp.float32) + 0.5 * nm
    d_ref[...] = d
    p_ref[...] = jnp.sum(d * d, axis=1, keepdims=True)[None]


def _normfin_kernel(p_ref, n_ref):
    s = jnp.sum(p_ref[...], axis=0)
    n_ref[...] = jnp.maximum(jnp.sqrt(s), 1e-12)


def _cosnum_kernel(a_ref, b_ref, up_ref, lo_ref, acc_ref):
    k = pl.program_id(1)

    @pl.when(k == 0)
    def _init():
        acc_ref[...] = jnp.zeros_like(acc_ref)

    acc_ref[...] += jnp.dot(a_ref[...], b_ref[...],
                            preferred_element_type=jnp.float32)

    @pl.when(k == pl.num_programs(1) - 1)
    def _fin():
        num = acc_ref[...]
        up_ref[...] = num
        lo_ref[...] = num.T


def _topk2_kernel(up_ref, lo_ref, ncol_ref, nrow_ref, vals_ref, idx_ref):
    i = pl.program_id(0)
    cols = jax.lax.broadcasted_iota(jnp.int32, (_BU, _U), 1)
    num = jnp.where(cols >= (i // 2) * _BC, up_ref[...], lo_ref[...])
    x = num / ncol_ref[...] / nrow_ref[...]
    for t in range(_K):
        m = jnp.max(x, axis=1, keepdims=True)
        first = jnp.min(jnp.where(x == m, cols, _U), axis=1, keepdims=True)
        onehot = cols == first
        vals_ref[:, t:t + 1] = m
        idx_ref[:, t:t + 1] = first
        x = jnp.where(onehot, -1.0, x)


def kernel(train_mat):
    f32 = jnp.float32
    tb = train_mat.astype(jnp.bfloat16)
    tbt = tb.T

    r = pl.pallas_call(
        _rowsum_kernel,
        grid=(_NB,),
        in_specs=[pl.BlockSpec((_BU, _I), lambda i: (i, 0))],
        out_specs=pl.BlockSpec((_BU, 1), lambda i: (i, 0)),
        out_shape=jax.ShapeDtypeStruct((_U, 1), f32),
    )(tb)
    rrow = r.reshape(1, _U)

    ntri8 = _NB * (_NB + 1) // 2

    def _ij8(t):
        return _tri_ij(t, _NB)

    jup, jlo = pl.pallas_call(
        _jacnum_kernel,
        grid=(ntri8,),
        in_specs=[
            pl.BlockSpec((_BU, _I), lambda t: (_ij8(t)[0], 0)),
            pl.BlockSpec((_I, _BU), lambda t: (0, _ij8(t)[1])),
        ],
        out_specs=[
            pl.BlockSpec((_BU, _BU), lambda t: _ij8(t)),
            pl.BlockSpec((_BU, _BU), lambda t: (_ij8(t)[1], _ij8(t)[0])),
        ],
        out_shape=[
            jax.ShapeDtypeStruct((_U, _U), f32),
            jax.ShapeDtypeStruct((_U, _U), f32),
        ],
    )(tb, tbt)

    w = pl.pallas_call(
        _topkw_kernel,
        grid=(_NB,),
        in_specs=[
            pl.BlockSpec((_BU, _U), lambda i: (i, 0)),
            pl.BlockSpec((_BU, _U), lambda i: (i, 0)),
            pl.BlockSpec((_BU, 1), lambda i: (i, 0)),
            pl.BlockSpec((1, _U), lambda i: (0, 0)),
        ],
        out_specs=pl.BlockSpec((_BU, _U), lambda i: (i, 0)),
        out_shape=jax.ShapeDtypeStruct((_U, _U), jnp.bfloat16),
    )(jup, jlo, r, rrow)

    d, p = pl.pallas_call(
        _d_kernel,
        grid=(8, _NB),
        in_specs=[
            pl.BlockSpec((_BU, _U), lambda j, i: (i, 0)),
            pl.BlockSpec((_U, _IC), lambda j, i: (0, j)),
            pl.BlockSpec((_BU, _IC), lambda j, i: (i, j)),
        ],
        out_specs=[
            pl.BlockSpec((_BU, _IC), lambda j, i: (i, j)),
            pl.BlockSpec((1, _BU, 1), lambda j, i: (j, i, 0)),
        ],
        out_shape=[
            jax.ShapeDtypeStruct((_U, _I), f32),
            jax.ShapeDtypeStruct((8, _U, 1), f32),
        ],
    )(w, tb, tb)

    n = pl.pallas_call(
        _normfin_kernel,
        grid=(1,),
        in_specs=[pl.BlockSpec((8, _U, 1), lambda i: (0, 0, 0))],
        out_specs=pl.BlockSpec((_U, 1), lambda i: (0, 0)),
        out_shape=jax.ShapeDtypeStruct((_U, 1), f32),
    )(p)
    nrow = n.reshape(1, _U)

    dt = d.T
    ntri4 = _NC * (_NC + 1) // 2

    def _ij4(t):
        return _tri_ij(t, _NC)

    cup, clo = pl.pallas_call(
        _cosnum_kernel,
        grid=(ntri4, 4),
        in_specs=[
            pl.BlockSpec((_BC, _KC), lambda t, k: (_ij4(t)[0], k)),
            pl.BlockSpec((_KC, _BC), lambda t, k: (k, _ij4(t)[1])),
        ],
        out_specs=[
            pl.BlockSpec((_BC, _BC), lambda t, k: _ij4(t)),
            pl.BlockSpec((_BC, _BC), lambda t, k: (_ij4(t)[1], _ij4(t)[0])),
        ],
        out_shape=[
            jax.ShapeDtypeStruct((_U, _U), f32),
            jax.ShapeDtypeStruct((_U, _U), f32),
        ],
        scratch_shapes=[pltpu.VMEM((_BC, _BC), f32)],
    )(d, dt)

    vals, idx = pl.pallas_call(
        _topk2_kernel,
        grid=(_NB,),
        in_specs=[
            pl.BlockSpec((_BU, _U), lambda i: (i, 0)),
            pl.BlockSpec((_BU, _U), lambda i: (i, 0)),
            pl.BlockSpec((_BU, 1), lambda i: (i, 0)),
            pl.BlockSpec((1, _U), lambda i: (0, 0)),
        ],
        out_specs=[
            pl.BlockSpec((_BU, _K), lambda i: (i, 0)),
            pl.BlockSpec((_BU, _K), lambda i: (i, 0)),
        ],
        out_shape=[
            jax.ShapeDtypeStruct((_U, _K), f32),
            jax.ShapeDtypeStruct((_U, _K), jnp.int32),
        ],
    )(cup, clo, n, nrow)
    return vals, idx
